# baseline (speedup 1.0000x reference)
import jax
import jax.numpy as jnp
from jax import lax
from jax.experimental import pallas as pl
from jax.experimental.pallas import tpu as pltpu
from jax.experimental.pallas import tpu_sc as plsc

NUM_USERS = 1000000
NUM_ITEMS = 1000000
EMB = 64
BATCH = 16384

NC = 2
NS = 16
NW = NC * NS
LANES = 16
B_PER_W = BATCH // NW
IDX_CHUNK = 128
N_IDX_CHUNKS = B_PER_W // IDX_CHUNK
ROW_PAIR = 2 * EMB
N_PASS = 2
B_PER_PASS = B_PER_W // N_PASS

_PARAMS = pltpu.CompilerParams(needs_layout_passes=False,
                               use_tc_tiling_on_sc=True)


def _fire_pairs(idx_v, flat_hbm, rows_v, sem, poff):
    def fire(j, _):
        start = pl.multiple_of(j * LANES, LANES)
        vec = lax.shift_right_logical(idx_v[pl.ds(poff + start, LANES)], 1)
        vec = vec * ROW_PAIR
        for l in range(LANES):
            dst = pl.ds((start + l) * ROW_PAIR, ROW_PAIR)
            o = pl.multiple_of(vec[l], ROW_PAIR)
            pltpu.async_copy(flat_hbm.at[pl.ds(o, ROW_PAIR)],
                             rows_v.at[dst], sem)
        return 0

    lax.fori_loop(0, B_PER_PASS // LANES, fire, 0)


def _drain_pairs(flat_hbm, rows_v, sem):
    def drain(e, _):
        pltpu.make_async_copy(flat_hbm.at[pl.ds(0, ROW_PAIR)],
                              rows_v.at[pl.ds(0, ROW_PAIR)], sem).wait()
        return 0

    lax.fori_loop(0, B_PER_PASS, drain, 0)


def _user_kernel_body(uidx_hbm, uflat_hbm, stage_hbm,
                      uidx_v, urows_v, stage_v, sem):
    wid = lax.axis_index("s") * NC + lax.axis_index("c")
    base = pl.multiple_of(wid * B_PER_W, B_PER_W)
    pltpu.sync_copy(uidx_hbm.at[pl.ds(base, B_PER_W)], uidx_v)

    for p in range(N_PASS):
        poff = p * B_PER_PASS
        _fire_pairs(uidx_v, uflat_hbm, urows_v, sem, poff)
        _drain_pairs(uflat_hbm, urows_v, sem)

        def chunk(j, _):
            start = pl.multiple_of(j * LANES, LANES)
            gs = pl.ds(poff + start, LANES)
            slotbase = (lax.iota(jnp.int32, LANES) + start) * ROW_PAIR
            ubase = slotbase + (uidx_v[gs] & 1) * EMB
            for d in range(EMB):
                stage_v[d, pl.ds(poff + start, LANES)] = (
                    plsc.load_gather(urows_v, [ubase + d]))
            return 0

        lax.fori_loop(0, B_PER_PASS // LANES, chunk, 0)

    pltpu.sync_copy(stage_v, stage_hbm.at[:, pl.ds(base, B_PER_W)])


def _item_kernel_body(uidx_hbm, iidx_hbm, iflat_hbm, ub_hbm, ib_hbm,
                      stage_hbm, out_hbm,
                      uidx_v, iidx_v, irows_v, stage_v, ubv, ibv, out_v, sem):
    wid = lax.axis_index("s") * NC + lax.axis_index("c")
    base = pl.multiple_of(wid * B_PER_W, B_PER_W)
    pltpu.sync_copy(uidx_hbm.at[pl.ds(base, B_PER_W)], uidx_v)
    pltpu.sync_copy(iidx_hbm.at[pl.ds(base, B_PER_W)], iidx_v)

    bias_copies = []
    for k in range(N_IDX_CHUNKS):
        s = pl.ds(k * IDX_CHUNK, IDX_CHUNK)
        bias_copies.append(pltpu.async_copy(ub_hbm.at[uidx_v.at[s]],
                                            ubv.at[s], sem))
        bias_copies.append(pltpu.async_copy(ib_hbm.at[iidx_v.at[s]],
                                            ibv.at[s], sem))

    stage_cp = pltpu.async_copy(stage_hbm.at[:, pl.ds(base, B_PER_W)],
                                stage_v, sem)

    for p in range(N_PASS):
        poff = p * B_PER_PASS
        _fire_pairs(iidx_v, iflat_hbm, irows_v, sem, poff)
        _drain_pairs(iflat_hbm, irows_v, sem)
        if p == 0:
            for c in bias_copies:
                c.wait()
            stage_cp.wait()

        def chunk(j, _):
            start = pl.multiple_of(j * LANES, LANES)
            gs = pl.ds(poff + start, LANES)
            slotbase = (lax.iota(jnp.int32, LANES) + start) * ROW_PAIR
            ibase = slotbase + (iidx_v[gs] & 1) * EMB
            acc = ubv[gs] + ibv[gs]
            for d in range(EMB):
                u = stage_v[d, gs]
                v = plsc.load_gather(irows_v, [ibase + d])
                acc = acc + u * v
            out_v[gs] = acc
            return 0

        lax.fori_loop(0, B_PER_PASS // LANES, chunk, 0)

    pltpu.sync_copy(out_v, out_hbm.at[pl.ds(base, B_PER_W)])


@jax.jit
def _run(uidx, iidx, uflat, iflat, ub, ib):
    mesh = plsc.VectorSubcoreMesh(core_axis_name="c", subcore_axis_name="s")
    ku = pl.kernel(
        _user_kernel_body, mesh=mesh,
        out_type=jax.ShapeDtypeStruct((EMB, BATCH), jnp.float32),
        scratch_types=[
            pltpu.VMEM((B_PER_W,), jnp.int32),
            pltpu.VMEM((B_PER_PASS * ROW_PAIR,), jnp.float32),
            pltpu.VMEM((EMB, B_PER_W), jnp.float32),
            pltpu.SemaphoreType.DMA,
        ],
        compiler_params=_PARAMS,
    )
    stage = ku(uidx, uflat)
    ki = pl.kernel(
        _item_kernel_body, mesh=mesh,
        out_type=jax.ShapeDtypeStruct((BATCH,), jnp.float32),
        scratch_types=[
            pltpu.VMEM((B_PER_W,), jnp.int32),
            pltpu.VMEM((B_PER_W,), jnp.int32),
            pltpu.VMEM((B_PER_PASS * ROW_PAIR,), jnp.float32),
            pltpu.VMEM((EMB, B_PER_W), jnp.float32),
            pltpu.VMEM((B_PER_W,), jnp.float32),
            pltpu.VMEM((B_PER_W,), jnp.float32),
            pltpu.VMEM((B_PER_W,), jnp.float32),
            pltpu.SemaphoreType.DMA,
        ],
        compiler_params=_PARAMS,
    )
    return ki(uidx, iidx, iflat, ub, ib, stage)


def kernel(user_idx, item_idx, user_embeddings, item_embeddings,
           user_biases, item_biases):
    uidx = user_idx.astype(jnp.int32)
    iidx = item_idx.astype(jnp.int32)
    uflat = jnp.reshape(user_embeddings, (NUM_USERS * EMB,))
    iflat = jnp.reshape(item_embeddings, (NUM_ITEMS * EMB,))
    ub = jnp.reshape(user_biases, (NUM_USERS,))
    ib = jnp.reshape(item_biases, (NUM_ITEMS,))
    return _run(uidx, iidx, uflat, iflat, ub, ib)


# speedup vs baseline: 1.0572x; 1.0572x over previous
import jax
import jax.numpy as jnp
from jax import lax
from jax.experimental import pallas as pl
from jax.experimental.pallas import tpu as pltpu
from jax.experimental.pallas import tpu_sc as plsc

NUM_USERS = 1000000
NUM_ITEMS = 1000000
EMB = 64
BATCH = 16384

NC = 2
NS = 16
NW = NC * NS
LANES = 16
B_PER_W = BATCH // NW
IDX_CHUNK = 128
N_IDX_CHUNKS = B_PER_W // IDX_CHUNK
ROW_PAIR = 2 * EMB
N_PASS = 2
B_PER_PASS = B_PER_W // N_PASS


def _sc_kernel_body(uidx_hbm, iidx_hbm, uflat_hbm, iflat_hbm, ub_hbm, ib_hbm,
                    out_hbm,
                    uidx_v, iidx_v, urows_v, irows_v,
                    ubv, ibv, out_v, sem):
    wid = lax.axis_index("s") * NC + lax.axis_index("c")
    base = pl.multiple_of(wid * B_PER_W, B_PER_W)

    pltpu.sync_copy(uidx_hbm.at[pl.ds(base, B_PER_W)], uidx_v)
    pltpu.sync_copy(iidx_hbm.at[pl.ds(base, B_PER_W)], iidx_v)

    bias_copies = []
    for k in range(N_IDX_CHUNKS):
        s = pl.ds(k * IDX_CHUNK, IDX_CHUNK)
        bias_copies.append(pltpu.async_copy(ub_hbm.at[uidx_v.at[s]],
                                            ubv.at[s], sem))
        bias_copies.append(pltpu.async_copy(ib_hbm.at[iidx_v.at[s]],
                                            ibv.at[s], sem))

    for p in range(N_PASS):
        poff = p * B_PER_PASS

        def fire(j, _):
            start = pl.multiple_of(j * LANES, LANES)
            gsl = pl.ds(poff + start, LANES)
            uvec = lax.shift_right_logical(uidx_v[gsl], 1) * ROW_PAIR
            ivec = lax.shift_right_logical(iidx_v[gsl], 1) * ROW_PAIR
            for l in range(LANES):
                dst = pl.ds((start + l) * ROW_PAIR, ROW_PAIR)
                uo = pl.multiple_of(uvec[l], ROW_PAIR)
                io = pl.multiple_of(ivec[l], ROW_PAIR)
                pltpu.async_copy(uflat_hbm.at[pl.ds(uo, ROW_PAIR)],
                                 urows_v.at[dst], sem)
                pltpu.async_copy(iflat_hbm.at[pl.ds(io, ROW_PAIR)],
                                 irows_v.at[dst], sem)
            return 0

        lax.fori_loop(0, B_PER_PASS // LANES, fire, 0)

        def drain(e, _):
            for _u in range(2):
                pltpu.make_async_copy(uflat_hbm.at[pl.ds(0, ROW_PAIR)],
                                      urows_v.at[pl.ds(0, ROW_PAIR)],
                                      sem).wait()
            return 0

        lax.fori_loop(0, B_PER_PASS, drain, 0)
        if p == 0:
            for c in bias_copies:
                c.wait()

        def chunk(j, _):
            start = pl.multiple_of(j * LANES, LANES)
            gs = pl.ds(poff + start, LANES)
            slotbase = (lax.iota(jnp.int32, LANES) + start) * ROW_PAIR
            ubase = slotbase + (uidx_v[gs] & 1) * EMB
            ibase = slotbase + (iidx_v[gs] & 1) * EMB
            acc = ubv[gs] + ibv[gs]
            for d in range(EMB):
                u = plsc.load_gather(urows_v, [ubase + d])
                v = plsc.load_gather(irows_v, [ibase + d])
                acc = acc + u * v
            out_v[gs] = acc
            return 0

        lax.fori_loop(0, B_PER_PASS // LANES, chunk, 0)

    pltpu.sync_copy(out_v, out_hbm.at[pl.ds(base, B_PER_W)])


@jax.jit
def _run(uidx, iidx, uemb, iemb, ub, ib):
    mesh = plsc.VectorSubcoreMesh(core_axis_name="c", subcore_axis_name="s")
    f = pl.kernel(
        _sc_kernel_body, mesh=mesh,
        out_type=jax.ShapeDtypeStruct((BATCH,), jnp.float32),
        scratch_types=[
            pltpu.VMEM((B_PER_W,), jnp.int32),
            pltpu.VMEM((B_PER_W,), jnp.int32),
            pltpu.VMEM((B_PER_PASS * ROW_PAIR,), jnp.float32),
            pltpu.VMEM((B_PER_PASS * ROW_PAIR,), jnp.float32),
            pltpu.VMEM((B_PER_W,), jnp.float32),
            pltpu.VMEM((B_PER_W,), jnp.float32),
            pltpu.VMEM((B_PER_W,), jnp.float32),
            pltpu.SemaphoreType.DMA,
        ],
        compiler_params=pltpu.CompilerParams(needs_layout_passes=False,
                                             use_tc_tiling_on_sc=True),
    )
    return f(uidx, iidx, uemb, iemb, ub, ib)


def kernel(user_idx, item_idx, user_embeddings, item_embeddings,
           user_biases, item_biases):
    uidx = user_idx.astype(jnp.int32)
    iidx = item_idx.astype(jnp.int32)
    uflat = jnp.reshape(user_embeddings, (NUM_USERS * EMB,))
    iflat = jnp.reshape(item_embeddings, (NUM_ITEMS * EMB,))
    ub = jnp.reshape(user_biases, (NUM_USERS,))
    ib = jnp.reshape(item_biases, (NUM_ITEMS,))
    return _run(uidx, iidx, uflat, iflat, ub, ib)
